# Initial kernel scaffold; baseline (speedup 1.0000x reference)
import jax
import jax.numpy as jnp
from jax.experimental import pallas as pl


def kernel(x, edge_index, batch, W0, b0, W1, b1, W2, b2):
    raise NotImplementedError("write your pallas kernel here")


# speedup vs baseline: 9.1792x; 9.1792x over previous
import functools

import jax
import jax.numpy as jnp
from jax import lax
from jax.experimental import pallas as pl
from jax.experimental.pallas import tpu as pltpu
from jax.experimental.pallas import tpu_sc as plsc

NC = 2
NS = 16
NW = NC * NS
LANES = 16
CHUNK = 128
GROUP = 4
N_ACC = 10240
DH = 64

_MESH = plsc.VectorSubcoreMesh(core_axis_name="c", subcore_axis_name="s")


def _row_agg_factory(n, k_chunks):
    rows_per_tile = N_ACC // NS
    assert rows_per_tile % 128 == 0

    @functools.partial(
        pl.kernel,
        out_type=jax.ShapeDtypeStruct((NC, N_ACC, DH), jnp.float32),
        mesh=_MESH,
        compiler_params=pltpu.CompilerParams(use_tc_tiling_on_sc=False),
        scratch_types=[
            pltpu.VMEM((k_chunks, CHUNK), jnp.int32),
            pltpu.VMEM((k_chunks, CHUNK), jnp.int32),
            pltpu.VMEM((GROUP, CHUNK, DH), jnp.float32),
            pltpu.VMEM((128, DH), jnp.float32),
            pltpu.VMEM_SHARED((N_ACC, DH), jnp.float32),
            pltpu.SemaphoreType.DMA,
            pltpu.SemaphoreType.DMA,
        ],
    )
    def agg(h_hbm, srcs_hbm, dsts_hbm, out_hbm, src_v, dst_v, rows, zeros_v,
            acc_sh, gsem, ssem):
        cid = lax.axis_index("c")
        sid = lax.axis_index("s")
        wid = sid * NC + cid

        zf = jnp.zeros((LANES,), jnp.float32)

        @pl.loop(0, 128)
        def _(r):
            for c in range(DH // LANES):
                zeros_v[r, pl.ds(c * LANES, LANES)] = zf

        base = sid * rows_per_tile
        for t in range(rows_per_tile // 128):
            pltpu.sync_copy(zeros_v, acc_sh.at[pl.ds(base + t * 128, 128)])
        plsc.subcore_barrier()

        pltpu.sync_copy(srcs_hbm.at[wid], src_v)
        pltpu.sync_copy(dsts_hbm.at[wid], dst_v)

        @pl.loop(0, k_chunks, step=GROUP)
        def _(g):
            descs = []
            for b in range(GROUP):
                descs.append(pltpu.async_copy(
                    h_hbm.at[src_v.at[g + b]], rows.at[b], gsem))
            for dsc in descs:
                dsc.wait()
            sdescs = []
            for b in range(GROUP):
                sdescs.append(pltpu.async_copy(
                    rows.at[b], acc_sh.at[dst_v.at[g + b]], ssem, add=True))
            for dsc in sdescs:
                dsc.wait()

        plsc.subcore_barrier()
        pltpu.sync_copy(acc_sh.at[pl.ds(base, rows_per_tile)],
                        out_hbm.at[cid, pl.ds(base, rows_per_tile)])

    return agg


def _scalar_agg_factory(n, k_chunks):
    per_tile = N_ACC // NS
    assert per_tile % LANES == 0

    @functools.partial(
        pl.kernel,
        out_type=jax.ShapeDtypeStruct((NC, N_ACC), jnp.float32),
        mesh=_MESH,
        scratch_types=[
            pltpu.VMEM((k_chunks, CHUNK), jnp.int32),
            pltpu.VMEM((k_chunks, CHUNK), jnp.int32),
            pltpu.VMEM((GROUP, CHUNK), jnp.float32),
            pltpu.VMEM((per_tile,), jnp.float32),
            pltpu.VMEM_SHARED((N_ACC,), jnp.float32),
            pltpu.SemaphoreType.DMA,
            pltpu.SemaphoreType.DMA,
        ],
    )
    def agg(vals_hbm, gidx_hbm, sidx_hbm, out_hbm, gidx_v, sidx_v, vals,
            zeros_v, acc_sh, gsem, ssem):
        cid = lax.axis_index("c")
        sid = lax.axis_index("s")
        wid = sid * NC + cid

        zf = jnp.zeros((LANES,), jnp.float32)

        @pl.loop(0, per_tile // LANES)
        def _(r):
            zeros_v[pl.ds(r * LANES, LANES)] = zf

        base = sid * per_tile
        pltpu.sync_copy(zeros_v, acc_sh.at[pl.ds(base, per_tile)])
        plsc.subcore_barrier()

        pltpu.sync_copy(gidx_hbm.at[wid], gidx_v)
        pltpu.sync_copy(sidx_hbm.at[wid], sidx_v)

        @pl.loop(0, k_chunks, step=GROUP)
        def _(g):
            descs = []
            for b in range(GROUP):
                descs.append(pltpu.async_copy(
                    vals_hbm.at[gidx_v.at[g + b]], vals.at[b], gsem))
            for dsc in descs:
                dsc.wait()
            sdescs = []
            for b in range(GROUP):
                sdescs.append(pltpu.async_copy(
                    vals.at[b], acc_sh.at[sidx_v.at[g + b]], ssem, add=True))
            for dsc in sdescs:
                dsc.wait()

        plsc.subcore_barrier()
        pltpu.sync_copy(acc_sh.at[pl.ds(base, per_tile)],
                        out_hbm.at[cid, pl.ds(base, per_tile)])

    return agg


def _k1(degp3, x, w0, bn):
    n, d_in = x.shape
    d_h = w0.shape[1]
    grid = n // bn

    def body(deg_ref, x_ref, w_ref, ha_ref, hb_ref, dis_ref):
        deg = deg_ref[0, :, 0] + deg_ref[1, :, 0] + 1.0
        dis = lax.rsqrt(deg)
        h = jnp.dot(x_ref[...], w_ref[...], preferred_element_type=jnp.float32)
        hp = h * dis[:, None]
        ha_ref[...] = hp[:, :DH]
        hb_ref[...] = hp[:, DH:]
        dis_ref[...] = dis[:, None]

    return pl.pallas_call(
        body,
        grid=(grid,),
        in_specs=[
            pl.BlockSpec((NC, bn, 1), lambda i: (0, i, 0)),
            pl.BlockSpec((bn, d_in), lambda i: (i, 0)),
            pl.BlockSpec((d_in, d_h), lambda i: (0, 0)),
        ],
        out_specs=[
            pl.BlockSpec((bn, DH), lambda i: (i, 0)),
            pl.BlockSpec((bn, DH), lambda i: (i, 0)),
            pl.BlockSpec((bn, 1), lambda i: (i, 0)),
        ],
        out_shape=[
            jax.ShapeDtypeStruct((n, DH), jnp.float32),
            jax.ShapeDtypeStruct((n, DH), jnp.float32),
            jax.ShapeDtypeStruct((n, 1), jnp.float32),
        ],
    )(degp3, x, w0)


def _k2(apa, apb, hpa, hpb, dis, b, w, bn, split_out):
    n = hpa.shape[0]
    d = 2 * DH
    d_out = w.shape[1]
    grid = n // bn

    def body(apa_ref, apb_ref, hpa_ref, hpb_ref, dis_ref, b_ref, w_ref, *outs):
        agg = jnp.concatenate(
            [apa_ref[0] + apa_ref[1] + hpa_ref[...],
             apb_ref[0] + apb_ref[1] + hpb_ref[...]], axis=1)
        o = jnp.maximum(dis_ref[...] * agg + b_ref[...][None, :], 0.0)
        h = jnp.dot(o, w_ref[...], preferred_element_type=jnp.float32)
        h = h * dis_ref[...]
        if split_out:
            outs[0][...] = h[:, :DH]
            outs[1][...] = h[:, DH:]
        else:
            outs[0][...] = h

    if split_out:
        out_specs = [pl.BlockSpec((bn, DH), lambda i: (i, 0)),
                     pl.BlockSpec((bn, DH), lambda i: (i, 0))]
        out_shape = [jax.ShapeDtypeStruct((n, DH), jnp.float32),
                     jax.ShapeDtypeStruct((n, DH), jnp.float32)]
    else:
        out_specs = [pl.BlockSpec((bn, d_out), lambda i: (i, 0))]
        out_shape = [jax.ShapeDtypeStruct((n, d_out), jnp.float32)]

    return pl.pallas_call(
        body,
        grid=(grid,),
        in_specs=[
            pl.BlockSpec((NC, bn, DH), lambda i: (0, i, 0)),
            pl.BlockSpec((NC, bn, DH), lambda i: (0, i, 0)),
            pl.BlockSpec((bn, DH), lambda i: (i, 0)),
            pl.BlockSpec((bn, DH), lambda i: (i, 0)),
            pl.BlockSpec((bn, 1), lambda i: (i, 0)),
            pl.BlockSpec((d,), lambda i: (0,)),
            pl.BlockSpec((d, d_out), lambda i: (0, 0)),
        ],
        out_specs=out_specs,
        out_shape=out_shape,
    )(apa, apb, hpa, hpb, dis, b, w)


def _k4(a2p3, h2p, dis, b2, batch2, n, g):
    def body(a2_ref, h2_ref, dis_ref, b2_ref, bat_ref, out_ref):
        a2 = a2_ref[0, :n, 0] + a2_ref[1, :n, 0]
        out2 = dis_ref[:, 0] * (a2 + h2_ref[:, 0]) + b2_ref[0]
        gid = bat_ref[:, 0]
        oh = (gid[:, None] == lax.broadcasted_iota(jnp.int32, (1, g), 1)
              ).astype(jnp.float32)
        sums = lax.dot_general(oh, out2[:, None],
                               (((0,), (0,)), ((), ())),
                               preferred_element_type=jnp.float32)
        counts = jnp.sum(oh, axis=0)
        mean = sums[:, 0] / jnp.maximum(counts, 1.0)
        out_ref[...] = 1.0 / (1.0 + jnp.exp(-mean))

    return pl.pallas_call(
        body,
        out_shape=jax.ShapeDtypeStruct((g,), jnp.float32),
    )(a2p3, h2p, dis, b2, batch2)


def kernel(x, edge_index, batch, W0, b0, W1, b1, W2, b2):
    n, d_in = x.shape
    e = edge_index.shape[1]
    g = 64
    bn = 2000

    k_chunks = -(-e // (NW * CHUNK))
    k_chunks = -(-k_chunks // GROUP) * GROUP
    e_pad = NW * k_chunks * CHUNK
    src = edge_index[0]
    dst = edge_index[1]
    srcs = jnp.concatenate(
        [src, jnp.zeros((e_pad - e,), jnp.int32)]).reshape(NW, k_chunks, CHUNK)
    dsts = jnp.concatenate(
        [dst, jnp.full((e_pad - e,), n, jnp.int32)]).reshape(NW, k_chunks, CHUNK)

    row_agg = _row_agg_factory(n, k_chunks)
    scalar_agg = _scalar_agg_factory(n, k_chunks)

    ones_pad = jnp.ones((N_ACC,), jnp.float32)
    degp = scalar_agg(ones_pad, dsts, dsts)

    h0a, h0b, dis = _k1(degp.reshape(NC, N_ACC, 1), x, W0, bn)
    a0a = row_agg(h0a, srcs, dsts)
    a0b = row_agg(h0b, srcs, dsts)
    h1a, h1b = _k2(a0a, a0b, h0a, h0b, dis, b0, W1, bn, True)
    a1a = row_agg(h1a, srcs, dsts)
    a1b = row_agg(h1b, srcs, dsts)
    h2p, = _k2(a1a, a1b, h1a, h1b, dis, b1, W2, bn, False)

    h2pad = jnp.concatenate([h2p[:, 0], jnp.zeros((N_ACC - n,), jnp.float32)])
    a2 = scalar_agg(h2pad, srcs, dsts)

    return _k4(a2.reshape(NC, N_ACC, 1), h2p, dis, b2,
               batch.reshape(n, 1), n, g)
